# baseline (speedup 1.0000x reference)
import functools

import jax
import jax.numpy as jnp
from jax import lax
from jax.experimental import pallas as pl
from jax.experimental.pallas import tpu as pltpu
from jax.experimental.pallas import tpu_sc as plsc

N_VOCAB = 1000000
DIM = 64
B = 16384
L = 50
EPS = 1e-5

NC = 2
NS = 16
NW = NC * NS
BAGS_W = B // NW
CHUNK = 4
ROWS = CHUNK * L
NCHUNK = BAGS_W // CHUNK
IDX_W = BAGS_W * L
NLANE = DIM // 16
GDIM = 128
ACC_BAGS = 128
CH_FLUSH = ACC_BAGS // CHUNK

TBLK = 512

_mesh = plsc.VectorSubcoreMesh(core_axis_name="c", subcore_axis_name="s")


def _tr_body(t_ref, o_ref):
    o_ref[:, :DIM] = t_ref[...].T


def _transpose_widen(table_t):
    return pl.pallas_call(
        _tr_body,
        grid=((N_VOCAB + TBLK - 1) // TBLK,),
        in_specs=[pl.BlockSpec((DIM, TBLK), lambda i: (0, i))],
        out_specs=pl.BlockSpec((TBLK, GDIM), lambda i: (i, 0)),
        out_shape=jax.ShapeDtypeStruct((N_VOCAB, GDIM), jnp.float32),
    )(table_t)


@functools.partial(
    pl.kernel,
    out_type=jax.ShapeDtypeStruct((B, DIM), jnp.float32),
    mesh=_mesh,
    scratch_types=[
        pltpu.VMEM((IDX_W,), jnp.int32),
        pltpu.VMEM((2, ROWS, GDIM), jnp.float32),
        pltpu.VMEM((ACC_BAGS, DIM), jnp.float32),
        pltpu.SemaphoreType.DMA,
        pltpu.SemaphoreType.DMA,
    ],
)
def _embed_pool(x_hbm, table_hbm, out_hbm, idx_v, rows_v, acc_v, sem0, sem1):
    wid = lax.axis_index("s") * NC + lax.axis_index("c")
    sems = (sem0, sem1)
    pltpu.sync_copy(x_hbm.at[pl.ds(wid * IDX_W, IDX_W)], idx_v)

    def gather(g, b):
        pltpu.make_async_copy(
            table_hbm.at[idx_v.at[pl.ds(g * ROWS, ROWS)]],
            rows_v.at[b],
            sems[b],
        ).start()

    def gwait(g, b):
        pltpu.make_async_copy(
            table_hbm.at[idx_v.at[pl.ds(g * ROWS, ROWS)]],
            rows_v.at[b],
            sems[b],
        ).wait()

    for b in range(2):
        gather(b, b)

    def outer_body(o, carry):
        for b in range(2):
            g = 2 * o + b
            gwait(g, b)
            buf = rows_v.at[b]
            arow0 = (g % CH_FLUSH) * CHUNK
            for bb in range(CHUNK):
                r0 = bb * L
                accs = [buf[r0, pl.ds(j * 16, 16)] for j in range(NLANE)]
                for r in range(1, L):
                    for j in range(NLANE):
                        accs[j] = accs[j] + buf[r0 + r, pl.ds(j * 16, 16)]
                for j in range(NLANE):
                    acc_v[arow0 + bb, pl.ds(j * 16, 16)] = accs[j]

            @pl.when(g + 2 < NCHUNK)
            def _():
                gather(g + 2, b)

            @pl.when(g % CH_FLUSH == CH_FLUSH - 1)
            def _():
                blk = g // CH_FLUSH
                pltpu.sync_copy(
                    acc_v,
                    out_hbm.at[pl.ds(wid * BAGS_W + blk * ACC_BAGS, ACC_BAGS)],
                )
        return carry

    lax.fori_loop(0, NCHUNK // 2, outer_body, 0)


def _mlp_body(p_ref, w1_ref, b1_ref, g_ref, be_ref, w2_ref, b2_ref, o_ref):
    p = p_ref[...]
    h = lax.dot_general(
        p, w1_ref[...], (((1,), (1,)), ((), ())),
        preferred_element_type=jnp.float32,
    ) + b1_ref[...]
    mu = jnp.mean(h, axis=0, keepdims=True)
    var = jnp.mean(jnp.square(h - mu), axis=0, keepdims=True)
    hn = (h - mu) * lax.rsqrt(var + EPS) * g_ref[...] + be_ref[...]
    hn = jnp.maximum(hn, 0.0)
    o_ref[...] = lax.dot_general(
        hn, w2_ref[...], (((1,), (1,)), ((), ())),
        preferred_element_type=jnp.float32,
    ) + b2_ref[...]


def kernel(x, table, W1, b1, gamma, beta, W2, b2):
    xflat = x.reshape(B * L).astype(jnp.int32)
    table_w = _transpose_widen(table.T)
    pooled = _embed_pool(xflat, table_w)
    return pl.pallas_call(
        _mlp_body,
        out_shape=jax.ShapeDtypeStruct((B, 4), jnp.float32),
    )(
        pooled,
        W1,
        b1.reshape(1, 32),
        gamma.reshape(1, 32),
        beta.reshape(1, 32),
        W2,
        b2.reshape(1, 4),
    )


# speedup vs baseline: 1.8392x; 1.8392x over previous
import functools

import jax
import jax.numpy as jnp
from jax import lax
from jax.experimental import pallas as pl
from jax.experimental.pallas import tpu as pltpu
from jax.experimental.pallas import tpu_sc as plsc

N_VOCAB = 1000000
DIM = 64
B = 16384
L = 50
EPS = 1e-5

NC = 2
NS = 16
NW = NC * NS
BAGS_W = B // NW
CHUNK = 4
ROWS = CHUNK * L
NCHUNK = BAGS_W // CHUNK
IDX_W = BAGS_W * L
NLANE = DIM // 16
GDIM = 128
ACC_BAGS = 128
CH_FLUSH = ACC_BAGS // CHUNK

_mesh = plsc.VectorSubcoreMesh(core_axis_name="c", subcore_axis_name="s")


@functools.partial(
    pl.kernel,
    out_type=jax.ShapeDtypeStruct((B, DIM), jnp.float32),
    mesh=_mesh,
    scratch_types=[
        pltpu.VMEM((IDX_W,), jnp.int32),
        pltpu.VMEM((2, ROWS, GDIM), jnp.float32),
        pltpu.VMEM((ACC_BAGS, DIM), jnp.float32),
        pltpu.SemaphoreType.DMA,
        pltpu.SemaphoreType.DMA,
    ],
)
def _embed_pool(x_hbm, table_hbm, out_hbm, idx_v, rows_v, acc_v, sem0, sem1):
    wid = lax.axis_index("s") * NC + lax.axis_index("c")
    sems = (sem0, sem1)
    pltpu.sync_copy(x_hbm.at[pl.ds(wid * IDX_W, IDX_W)], idx_v)

    def gather(g, b):
        pltpu.make_async_copy(
            table_hbm.at[idx_v.at[pl.ds(g * ROWS, ROWS)]],
            rows_v.at[b],
            sems[b],
        ).start()

    def gwait(g, b):
        pltpu.make_async_copy(
            table_hbm.at[idx_v.at[pl.ds(g * ROWS, ROWS)]],
            rows_v.at[b],
            sems[b],
        ).wait()

    for b in range(2):
        gather(b, b)

    def outer_body(o, carry):
        for b in range(2):
            g = 2 * o + b
            gwait(g, b)
            buf = rows_v.at[b]
            arow0 = (g % CH_FLUSH) * CHUNK
            for bb in range(CHUNK):
                r0 = bb * L
                accs = [buf[r0, pl.ds(j * 16, 16)] for j in range(NLANE)]
                for r in range(1, L):
                    for j in range(NLANE):
                        accs[j] = accs[j] + buf[r0 + r, pl.ds(j * 16, 16)]
                for j in range(NLANE):
                    acc_v[arow0 + bb, pl.ds(j * 16, 16)] = accs[j]

            @pl.when(g + 2 < NCHUNK)
            def _():
                gather(g + 2, b)

            @pl.when(g % CH_FLUSH == CH_FLUSH - 1)
            def _():
                blk = g // CH_FLUSH
                pltpu.sync_copy(
                    acc_v,
                    out_hbm.at[pl.ds(wid * BAGS_W + blk * ACC_BAGS, ACC_BAGS)],
                )
        return carry

    lax.fori_loop(0, NCHUNK // 2, outer_body, 0)


def _mlp_body(p_ref, w1_ref, b1_ref, g_ref, be_ref, w2_ref, b2_ref, o_ref):
    p = p_ref[...]
    h = lax.dot_general(
        p, w1_ref[...], (((1,), (1,)), ((), ())),
        preferred_element_type=jnp.float32,
    ) + b1_ref[...]
    mu = jnp.mean(h, axis=0, keepdims=True)
    var = jnp.mean(jnp.square(h - mu), axis=0, keepdims=True)
    hn = (h - mu) * lax.rsqrt(var + EPS) * g_ref[...] + be_ref[...]
    hn = jnp.maximum(hn, 0.0)
    o_ref[...] = lax.dot_general(
        hn, w2_ref[...], (((1,), (1,)), ((), ())),
        preferred_element_type=jnp.float32,
    ) + b2_ref[...]


def kernel(x, table, W1, b1, gamma, beta, W2, b2):
    xflat = x.reshape(B * L).astype(jnp.int32)
    table_w = jnp.pad(table, ((0, 0), (0, GDIM - DIM)))
    pooled = _embed_pool(xflat, table_w)
    return pl.pallas_call(
        _mlp_body,
        out_shape=jax.ShapeDtypeStruct((B, 4), jnp.float32),
    )(
        pooled,
        W1,
        b1.reshape(1, 32),
        gamma.reshape(1, 32),
        beta.reshape(1, 32),
        W2,
        b2.reshape(1, 4),
    )
